# baseline (speedup 1.0000x reference)
import jax
import jax.numpy as jnp
from jax import lax
from jax.experimental import pallas as pl
from jax.experimental.pallas import tpu as pltpu
from jax.experimental.pallas import tpu_sc as plsc

_B = 16384
_V = 1000000
_D = 64
_BN_EPS = 1e-5

_NC = 2
_NS = 16
_NW = _NC * _NS
_CW = 512
_NCHUNKS = -(-_V // _CW)
_CAP = 768
_NG = _B // 16
_VPAD = 1000064
_NBK = 8
_BKC = 128
_TPW = -(-_NCHUNKS // _NW)
_BKSPAN = -(-_TPW // _NBK)


def _sc_body(ui_hbm, ii_hbm, ugT, umT, igT, imT,
             ug_o, um_o, ig_o, im_o,
             idx_v, cl_idx, cl_pos, wk_idx, wk_pos,
             buf, stage, bk_idx, bk_pos, bcnt_s, sem_f, sem_s):
    wid = lax.axis_index("s") * _NC + lax.axis_index("c")
    lanes = lax.iota(jnp.int32, 16)
    n_my = (_NCHUNKS - wid + _NW - 1) // _NW

    def build_lists(src_idx):
        pltpu.sync_copy(src_idx, idx_v)

        def scan(g, off):
            row = g // 8
            sub = (g % 8) * 16
            iv = idx_v[row, pl.ds(sub, 16)]
            m = ((iv // _CW) % _NW) == wid
            plsc.store_compressed(cl_idx.at[pl.ds(off, 16)], iv, mask=m)
            plsc.store_compressed(cl_pos.at[pl.ds(off, 16)],
                                  g * 16 + lanes, mask=m)
            return off + jnp.sum(m.astype(jnp.int32))

        cnt = lax.fori_loop(0, _NG, scan, 0)
        ngrp = (cnt + 15) // 16
        for k in range(_NBK):
            def bscan(g, offk):
                iv = cl_idx[pl.ds(g * 16, 16)]
                pv = cl_pos[pl.ds(g * 16, 16)]
                valid = (g * 16 + lanes) < cnt
                t_of = (iv // _CW - wid) // _NW
                m = ((t_of // _BKSPAN) == k) & valid
                plsc.store_compressed(
                    bk_idx.at[pl.ds(k * _BKC + offk, 16)], iv, mask=m)
                plsc.store_compressed(
                    bk_pos.at[pl.ds(k * _BKC + offk, 16)], pv, mask=m)
                return offk + jnp.sum(m.astype(jnp.int32))

            bcnt_s[k] = lax.fori_loop(0, ngrp, bscan, 0)

    def do_pass(tbl, out):
        def rescan(t, p):
            c = t * _NW + wid
            k = t // _BKSPAN
            kb = k * _BKC
            nk = bcnt_s[k]

            def body(g, off2):
                iv = bk_idx[pl.ds(kb + g * 16, 16)]
                pv = bk_pos[pl.ds(kb + g * 16, 16)]
                valid = (g * 16 + lanes) < nk
                m = ((iv // _CW) == c) & valid
                plsc.store_compressed(
                    wk_idx.at[pl.ds(p * _CAP + off2, 16)], iv, mask=m)
                plsc.store_compressed(
                    wk_pos.at[pl.ds(p * _CAP + off2, 16)], pv, mask=m)
                return off2 + jnp.sum(m.astype(jnp.int32))

            return lax.fori_loop(0, (nk + 15) // 16, body, 0)

        def lo_of(t):
            return jnp.minimum((t * _NW + wid) * _CW, _VPAD - _CW)

        def fire(t, p, c2):
            lo = lo_of(t)

            @pl.when(c2 > 0)
            def _():
                pltpu.async_copy(tbl.at[:, pl.ds(lo, _CW)],
                                 buf.at[pl.ds(p * _D, _D)], sem_f)

        def extract(t, p, c2):
            lo = lo_of(t)

            @pl.when(c2 > 0)
            def _():
                pltpu.make_async_copy(tbl.at[:, pl.ds(lo, _CW)],
                                      buf.at[pl.ds(p * _D, _D)], sem_f).wait()

            def group(g2, _):
                base = p * _CAP + g2 * 16
                valid = (g2 * 16 + lanes) < c2
                iv = wk_idx[pl.ds(base, 16)]
                pv = wk_pos[pl.ds(base, 16)]
                col = jnp.where(valid, iv - lo, 0)
                pos = jnp.where(valid, pv, _B + lanes)
                for d in range(_D):
                    dsp = jnp.full((16,), d, jnp.int32)
                    v = plsc.load_gather(buf, [dsp + p * _D, col])
                    plsc.store_scatter(stage, [lanes, dsp], v)
                pltpu.async_copy(stage, out.at[pos], sem_s).wait()
                return 0

            lax.fori_loop(0, (c2 + 15) // 16, group, 0)

        c2_0 = rescan(0, 0)
        fire(0, 0, c2_0)

        def step(t, c2p):
            p = t % 2
            c2 = rescan(t, p)
            fire(t, p, c2)
            extract(t - 1, (t - 1) % 2, c2p)
            return c2

        c2_last = lax.fori_loop(1, n_my, step, c2_0)
        extract(n_my - 1, (n_my - 1) % 2, c2_last)

    build_lists(ui_hbm)
    do_pass(ugT, ug_o)
    do_pass(umT, um_o)
    build_lists(ii_hbm)
    do_pass(igT, ig_o)
    do_pass(imT, im_o)


def _sc_gather(ui, ii, ugT, umT, igT, imT):
    mesh = plsc.VectorSubcoreMesh(core_axis_name="c", subcore_axis_name="s")
    out_t = jax.ShapeDtypeStruct((_B + 16, 2 * _D), jnp.float32)
    fn = pl.kernel(
        _sc_body,
        out_type=[out_t, out_t, out_t, out_t],
        mesh=mesh,
        scratch_types=[
            pltpu.VMEM((_B // 128, 128), jnp.int32),
            pltpu.VMEM((_CAP,), jnp.int32),
            pltpu.VMEM((_CAP,), jnp.int32),
            pltpu.VMEM((2 * _CAP,), jnp.int32),
            pltpu.VMEM((2 * _CAP,), jnp.int32),
            pltpu.VMEM((2 * _D, _CW), jnp.float32),
            pltpu.VMEM((16, 2 * _D), jnp.float32),
            pltpu.VMEM((_NBK * _BKC + 16,), jnp.int32),
            pltpu.VMEM((_NBK * _BKC + 16,), jnp.int32),
            pltpu.SMEM((_NBK,), jnp.int32),
            pltpu.SemaphoreType.DMA,
            pltpu.SemaphoreType.DMA,
        ],
        compiler_params=pltpu.CompilerParams(use_tc_tiling_on_sc=True,
                                             needs_layout_passes=False),
    )
    return fn(ui.reshape(_B // 128, 128), ii.reshape(_B // 128, 128),
              ugT, umT, igT, imT)


_BLK = 2048


def _tc_body(ug, um, ig, im, W0, b0, g0, be0, W1, b1, g1, be1,
             W2, b2, g2, be2, wout, bout, out_ref):
    inv = 1.0 / (1.0 + _BN_EPS) ** 0.5
    gmf = ug[:, : _D] * ig[:, : _D]
    x = jnp.concatenate([um[:, : _D], im[:, : _D]], axis=1)
    for W, b, g, be in ((W0, b0, g0, be0), (W1, b1, g1, be1),
                        (W2, b2, g2, be2)):
        h = jnp.dot(x, W[...], preferred_element_type=jnp.float32) + b[...]
        h = jnp.maximum(h, 0.0)
        x = h * (g[...] * inv) + be[...]
    pred = jnp.concatenate([gmf, x], axis=1)
    logits = jnp.sum(pred * wout[...], axis=1, keepdims=True) + bout[...]
    out_ref[...] = 1.0 / (1.0 + jnp.exp(-logits))


def _tc_dense(ug, um, ig, im, W0, b0, g0, be0, W1, b1, g1, be1,
              W2, b2, g2, be2, wout_row, bout):
    grid = _B // _BLK
    row_spec = pl.BlockSpec((_BLK, 2 * _D), lambda i: (i, 0))

    def full(shape):
        return pl.BlockSpec(shape, lambda i: tuple(0 for _ in shape))

    return pl.pallas_call(
        _tc_body,
        grid=(grid,),
        in_specs=[
            row_spec, row_spec, row_spec, row_spec,
            full(W0.shape), full(b0.shape), full(g0.shape), full(be0.shape),
            full(W1.shape), full(b1.shape), full(g1.shape), full(be1.shape),
            full(W2.shape), full(b2.shape), full(g2.shape), full(be2.shape),
            full(wout_row.shape), full(bout.shape),
        ],
        out_specs=pl.BlockSpec((_BLK, 1), lambda i: (i, 0)),
        out_shape=jax.ShapeDtypeStruct((_B, 1), jnp.float32),
    )(ug, um, ig, im, W0, b0, g0, be0, W1, b1, g1, be1,
      W2, b2, g2, be2, wout_row, bout)


def kernel(user_indices, item_indices, user_gmf, item_gmf, user_mlp, item_mlp,
           W0, b0, g0, be0, W1, b1, g1, be1, W2, b2, g2, be2, Wout, bout):
    ui = user_indices.astype(jnp.int32)
    ii = item_indices.astype(jnp.int32)
    ug_o, um_o, ig_o, im_o = _sc_gather(ui, ii, user_gmf.T, user_mlp.T,
                                        item_gmf.T, item_mlp.T)
    return _tc_dense(
        ug_o, um_o, ig_o, im_o,
        W0, b0.reshape(1, -1), g0.reshape(1, -1), be0.reshape(1, -1),
        W1, b1.reshape(1, -1), g1.reshape(1, -1), be1.reshape(1, -1),
        W2, b2.reshape(1, -1), g2.reshape(1, -1), be2.reshape(1, -1),
        Wout.reshape(1, -1), bout.reshape(1, 1))


# speedup vs baseline: 1.3917x; 1.3917x over previous
import jax
import jax.numpy as jnp
from jax import lax
from jax.experimental import pallas as pl
from jax.experimental.pallas import tpu as pltpu
from jax.experimental.pallas import tpu_sc as plsc

_B = 16384
_V = 1000000
_D = 64
_BN_EPS = 1e-5

_NC = 2
_NS = 16
_NW = _NC * _NS
_CW = 512
_NCHUNKS = -(-_V // _CW)
_CAP = 768
_NG = _B // 16
_VPAD = 1000064
_NBK = 8
_BKC = 128
_TPW = -(-_NCHUNKS // _NW)
_BKSPAN = -(-_TPW // _NBK)
_RING = 8


def _sc_body(ui_hbm, ii_hbm, ugT, umT, igT, imT, out_u, out_i,
             uidx_v, iidx_v, cl_idx, cl_pos, wk_idx, wk_pos,
             bg, bm, stage, bk_idx, bk_pos, bcnt_s,
             sem_g, sem_m, sem_s):
    wid = lax.axis_index("s") * _NC + lax.axis_index("c")
    pltpu.sync_copy(ui_hbm, uidx_v)
    pltpu.sync_copy(ii_hbm, iidx_v)
    lanes = lax.iota(jnp.int32, 16)

    def build_list(idx_v):
        def scan(g, off):
            row = g // 8
            sub = (g % 8) * 16
            iv = idx_v[row, pl.ds(sub, 16)]
            m = ((iv // _CW) % _NW) == wid
            plsc.store_compressed(cl_idx.at[pl.ds(off, 16)], iv, mask=m)
            plsc.store_compressed(cl_pos.at[pl.ds(off, 16)],
                                  g * 16 + lanes, mask=m)
            return off + jnp.sum(m.astype(jnp.int32))

        return lax.fori_loop(0, _NG, scan, 0)

    def do_phase(idx_v, tg, tm, out):
        cnt = build_list(idx_v)
        ngrp = (cnt + 15) // 16

        for k in range(_NBK):
            def bscan(g, offk):
                iv = cl_idx[pl.ds(g * 16, 16)]
                pv = cl_pos[pl.ds(g * 16, 16)]
                valid = (g * 16 + lanes) < cnt
                t_of = (iv // _CW - wid) // _NW
                m = ((t_of // _BKSPAN) == k) & valid
                plsc.store_compressed(
                    bk_idx.at[pl.ds(k * _BKC + offk, 16)], iv, mask=m)
                plsc.store_compressed(
                    bk_pos.at[pl.ds(k * _BKC + offk, 16)], pv, mask=m)
                return offk + jnp.sum(m.astype(jnp.int32))

            bcnt_s[k] = lax.fori_loop(0, ngrp, bscan, 0)

        n_my = (_NCHUNKS - wid + _NW - 1) // _NW

        def chunk_body(t, gc0):
            c = t * _NW + wid
            lo = jnp.minimum(c * _CW, _VPAD - _CW)
            cp_g = pltpu.async_copy(tg.at[:, pl.ds(lo, _CW)], bg, sem_g)
            cp_m = pltpu.async_copy(tm.at[:, pl.ds(lo, _CW)], bm, sem_m)

            k = t // _BKSPAN
            kb = k * _BKC
            nk = bcnt_s[k]

            def rescan(g, off2):
                iv = bk_idx[pl.ds(kb + g * 16, 16)]
                pv = bk_pos[pl.ds(kb + g * 16, 16)]
                valid = (g * 16 + lanes) < nk
                m = ((iv // _CW) == c) & valid
                plsc.store_compressed(wk_idx.at[pl.ds(off2, 16)], iv, mask=m)
                plsc.store_compressed(wk_pos.at[pl.ds(off2, 16)], pv, mask=m)
                return off2 + jnp.sum(m.astype(jnp.int32))

            cnt2 = lax.fori_loop(0, (nk + 15) // 16, rescan, 0)
            cp_g.wait()
            cp_m.wait()

            def extract(g2, _):
                gc = gc0 + g2
                slot = (gc % _RING) * 16

                @pl.when(gc >= _RING)
                def _():
                    pltpu.make_async_copy(
                        stage.at[pl.ds(0, 16)], out.at[pl.ds(0, 16)],
                        sem_s).wait()

                base = g2 * 16
                valid = (base + lanes) < cnt2
                iv = wk_idx[pl.ds(base, 16)]
                pv = wk_pos[pl.ds(base, 16)]
                col = jnp.where(valid, iv - lo, 0)
                pos = jnp.where(valid, pv, _B + lanes)
                for d in range(_D):
                    dsp = jnp.full((16,), d, jnp.int32)
                    vg = plsc.load_gather(bg, [dsp, col])
                    plsc.store_scatter(stage, [slot + lanes, dsp], vg)
                    vm = plsc.load_gather(bm, [dsp, col])
                    plsc.store_scatter(stage, [slot + lanes, dsp + _D], vm)
                pltpu.async_copy(stage.at[pl.ds(slot, 16)], out.at[pos],
                                 sem_s)
                return 0

            ng2 = (cnt2 + 15) // 16
            lax.fori_loop(0, ng2, extract, 0)
            return gc0 + ng2

        gc_end = lax.fori_loop(0, n_my, chunk_body, 0)

        def drain(_i, _):
            pltpu.make_async_copy(stage.at[pl.ds(0, 16)],
                                  out.at[pl.ds(0, 16)], sem_s).wait()
            return 0

        lax.fori_loop(0, jnp.minimum(gc_end, _RING), drain, 0)

    do_phase(uidx_v, ugT, umT, out_u)
    do_phase(iidx_v, igT, imT, out_i)


def _sc_gather(ui, ii, ugT, umT, igT, imT):
    mesh = plsc.VectorSubcoreMesh(core_axis_name="c", subcore_axis_name="s")
    out_t = jax.ShapeDtypeStruct((_B + 16, 2 * _D), jnp.float32)
    fn = pl.kernel(
        _sc_body,
        out_type=[out_t, out_t],
        mesh=mesh,
        scratch_types=[
            pltpu.VMEM((_B // 128, 128), jnp.int32),
            pltpu.VMEM((_B // 128, 128), jnp.int32),
            pltpu.VMEM((_CAP,), jnp.int32),
            pltpu.VMEM((_CAP,), jnp.int32),
            pltpu.VMEM((_CAP,), jnp.int32),
            pltpu.VMEM((_CAP,), jnp.int32),
            pltpu.VMEM((_D, _CW), jnp.float32),
            pltpu.VMEM((_D, _CW), jnp.float32),
            pltpu.VMEM((_RING * 16, 2 * _D), jnp.float32),
            pltpu.VMEM((_NBK * _BKC + 16,), jnp.int32),
            pltpu.VMEM((_NBK * _BKC + 16,), jnp.int32),
            pltpu.SMEM((_NBK,), jnp.int32),
            pltpu.SemaphoreType.DMA,
            pltpu.SemaphoreType.DMA,
            pltpu.SemaphoreType.DMA,
        ],
        compiler_params=pltpu.CompilerParams(use_tc_tiling_on_sc=True,
                                             needs_layout_passes=False),
    )
    return fn(ui.reshape(_B // 128, 128), ii.reshape(_B // 128, 128),
              ugT, umT, igT, imT)


_BLK = 2048


def _tc_body(uo, io, W0, b0, g0, be0, W1, b1, g1, be1,
             W2, b2, g2, be2, wout, bout, out_ref):
    inv = 1.0 / (1.0 + _BN_EPS) ** 0.5
    u = uo[...]
    i = io[...]
    gmf = u[:, :_D] * i[:, :_D]
    x = jnp.concatenate([u[:, _D:], i[:, _D:]], axis=1)
    for W, b, g, be in ((W0, b0, g0, be0), (W1, b1, g1, be1),
                        (W2, b2, g2, be2)):
        h = jnp.dot(x, W[...], preferred_element_type=jnp.float32) + b[...]
        h = jnp.maximum(h, 0.0)
        x = h * (g[...] * inv) + be[...]
    pred = jnp.concatenate([gmf, x], axis=1)
    logits = jnp.sum(pred * wout[...], axis=1, keepdims=True) + bout[...]
    out_ref[...] = 1.0 / (1.0 + jnp.exp(-logits))


def _tc_dense(uo, io, W0, b0, g0, be0, W1, b1, g1, be1,
              W2, b2, g2, be2, wout_row, bout):
    grid = _B // _BLK
    row_spec = pl.BlockSpec((_BLK, 2 * _D), lambda i: (i, 0))

    def full(shape):
        return pl.BlockSpec(shape, lambda i: tuple(0 for _ in shape))

    return pl.pallas_call(
        _tc_body,
        grid=(grid,),
        in_specs=[
            row_spec, row_spec,
            full(W0.shape), full(b0.shape), full(g0.shape), full(be0.shape),
            full(W1.shape), full(b1.shape), full(g1.shape), full(be1.shape),
            full(W2.shape), full(b2.shape), full(g2.shape), full(be2.shape),
            full(wout_row.shape), full(bout.shape),
        ],
        out_specs=pl.BlockSpec((_BLK, 1), lambda i: (i, 0)),
        out_shape=jax.ShapeDtypeStruct((_B, 1), jnp.float32),
    )(uo, io, W0, b0, g0, be0, W1, b1, g1, be1,
      W2, b2, g2, be2, wout_row, bout)


def kernel(user_indices, item_indices, user_gmf, item_gmf, user_mlp, item_mlp,
           W0, b0, g0, be0, W1, b1, g1, be1, W2, b2, g2, be2, Wout, bout):
    ui = user_indices.astype(jnp.int32)
    ii = item_indices.astype(jnp.int32)
    out_u, out_i = _sc_gather(ui, ii, user_gmf.T, user_mlp.T,
                              item_gmf.T, item_mlp.T)
    return _tc_dense(
        out_u, out_i,
        W0, b0.reshape(1, -1), g0.reshape(1, -1), be0.reshape(1, -1),
        W1, b1.reshape(1, -1), g1.reshape(1, -1), be1.reshape(1, -1),
        W2, b2.reshape(1, -1), g2.reshape(1, -1), be2.reshape(1, -1),
        Wout.reshape(1, -1), bout.reshape(1, 1))
